# baseline (speedup 1.0000x reference)
import functools

import jax
import jax.numpy as jnp
from jax import lax
from jax.experimental import pallas as pl
from jax.experimental.pallas import tpu as pltpu
from jax.experimental.pallas import tpu_sc as plsc


def _sc_gather(table, idx):
    B = idx.shape[0]
    V, D = table.shape
    info = plsc.get_sparse_core_info()
    nw = info.num_cores * info.num_subcores
    b_per_w = B // nw
    mesh = plsc.VectorSubcoreMesh(core_axis_name="c", subcore_axis_name="s")

    @functools.partial(
        pl.kernel,
        mesh=mesh,
        out_type=jax.ShapeDtypeStruct((B, D), jnp.float32),
        scratch_types=[
            pltpu.VMEM((b_per_w,), jnp.int32),
            pltpu.VMEM((b_per_w, D), jnp.float32),
            pltpu.SemaphoreType.DMA,
        ],
    )
    def gather_kernel(table_hbm, idx_hbm, out_hbm, idx_v, rows_v, sem):
        wid = lax.axis_index("s") * info.num_cores + lax.axis_index("c")
        base = wid * b_per_w
        pltpu.sync_copy(idx_hbm.at[pl.ds(base, b_per_w)], idx_v)
        pltpu.async_copy(table_hbm.at[idx_v], rows_v, sem).wait()
        pltpu.sync_copy(rows_v, out_hbm.at[pl.ds(base, b_per_w)])

    return gather_kernel(table, idx)


def _tc_project(embeds, W_out, b_out, tile_v=2048, nbuf=4):
    B, D = embeds.shape
    V = W_out.shape[0]
    nv = pl.cdiv(V, tile_v)
    rem = V - (nv - 1) * tile_v
    b2 = b_out.reshape(1, V)

    def body(e_ref, w_ref, b_ref, o_hbm, obuf, olast, sems, lsem):
        i = pl.program_id(0)
        slot = lax.rem(i, nbuf)

        @pl.when(i >= nbuf)
        def _():
            pltpu.make_async_copy(
                obuf.at[slot],
                o_hbm.at[:, pl.ds((i - nbuf) * tile_v, tile_v)],
                sems.at[slot],
            ).wait()

        acc = lax.dot_general(
            e_ref[...], w_ref[...],
            dimension_numbers=(((1,), (1,)), ((), ())),
            preferred_element_type=jnp.float32,
        ) + b_ref[...]

        @pl.when(i < nv - 1)
        def _():
            obuf[slot] = acc
            pltpu.make_async_copy(
                obuf.at[slot], o_hbm.at[:, pl.ds(i * tile_v, tile_v)],
                sems.at[slot],
            ).start()

        @pl.when(i == nv - 1)
        def _():
            olast[...] = acc[:, :rem]
            last = pltpu.make_async_copy(
                olast, o_hbm.at[:, pl.ds((nv - 1) * tile_v, rem)], lsem,
            )
            last.start()
            for k in range(nbuf - 1, 0, -1):
                j = nv - 1 - k
                if j >= 0:
                    pltpu.make_async_copy(
                        obuf.at[j % nbuf],
                        o_hbm.at[:, pl.ds(j * tile_v, tile_v)],
                        sems.at[j % nbuf],
                    ).wait()
            last.wait()

    return pl.pallas_call(
        body,
        grid=(nv,),
        in_specs=[
            pl.BlockSpec((B, D), lambda i: (0, 0)),
            pl.BlockSpec((tile_v, D), lambda i: (i, 0)),
            pl.BlockSpec((1, tile_v), lambda i: (0, i)),
        ],
        out_specs=pl.BlockSpec(memory_space=pl.ANY),
        scratch_shapes=[
            pltpu.VMEM((nbuf, B, tile_v), jnp.float32),
            pltpu.VMEM((B, rem), jnp.float32),
            pltpu.SemaphoreType.DMA((nbuf,)),
            pltpu.SemaphoreType.DMA,
        ],
        out_shape=jax.ShapeDtypeStruct((B, V), jnp.float32),
    )(embeds, W_out, b2)


def kernel(input_word_ids, in_table, W_out, b_out):
    ids = input_word_ids.astype(jnp.int32)
    embeds = _sc_gather(in_table, ids)
    return _tc_project(embeds, W_out, b_out)


# speedup vs baseline: 2.3319x; 2.3319x over previous
import functools

import jax
import jax.numpy as jnp
from jax import lax
from jax.experimental import pallas as pl
from jax.experimental.pallas import tpu as pltpu
from jax.experimental.pallas import tpu_sc as plsc


def _sc_gather(table, idx):
    B = idx.shape[0]
    V, D = table.shape
    info = plsc.get_sparse_core_info()
    nw = info.num_cores * info.num_subcores
    b_per_w = B // nw
    mesh = plsc.VectorSubcoreMesh(core_axis_name="c", subcore_axis_name="s")

    @functools.partial(
        pl.kernel,
        mesh=mesh,
        out_type=jax.ShapeDtypeStruct((B, D), jnp.float32),
        scratch_types=[
            pltpu.VMEM((b_per_w,), jnp.int32),
            pltpu.VMEM((b_per_w, D), jnp.float32),
            pltpu.SemaphoreType.DMA,
        ],
    )
    def gather_kernel(table_hbm, idx_hbm, out_hbm, idx_v, rows_v, sem):
        wid = lax.axis_index("s") * info.num_cores + lax.axis_index("c")
        base = wid * b_per_w
        pltpu.sync_copy(idx_hbm.at[pl.ds(base, b_per_w)], idx_v)
        pltpu.async_copy(table_hbm.at[idx_v], rows_v, sem).wait()
        pltpu.sync_copy(rows_v, out_hbm.at[pl.ds(base, b_per_w)])

    return gather_kernel(table, idx)


def _tc_project_t(embeds, W_out, b_out, tile_v=2000):
    B, D = embeds.shape
    V = W_out.shape[0]
    nv = pl.cdiv(V, tile_v)
    b2 = b_out.reshape(V, 1)

    def body(e_ref, w_ref, b_ref, o_ref):
        acc = lax.dot_general(
            w_ref[...], e_ref[...],
            dimension_numbers=(((1,), (1,)), ((), ())),
            preferred_element_type=jnp.float32,
        )
        o_ref[...] = acc + b_ref[...]

    return pl.pallas_call(
        body,
        grid=(nv,),
        in_specs=[
            pl.BlockSpec((B, D), lambda i: (0, 0)),
            pl.BlockSpec((tile_v, D), lambda i: (i, 0)),
            pl.BlockSpec((tile_v, 1), lambda i: (i, 0)),
        ],
        out_specs=pl.BlockSpec((tile_v, B), lambda i: (i, 0)),
        out_shape=jax.ShapeDtypeStruct((V, B), jnp.float32),
    )(embeds, W_out, b2)


def kernel(input_word_ids, in_table, W_out, b_out):
    ids = input_word_ids.astype(jnp.int32)
    embeds = _sc_gather(in_table, ids)
    return _tc_project_t(embeds, W_out, b_out).T


# speedup vs baseline: 3.1447x; 1.3485x over previous
import functools

import jax
import jax.numpy as jnp
from jax import lax
from jax.experimental import pallas as pl
from jax.experimental.pallas import tpu as pltpu
from jax.experimental.pallas import tpu_sc as plsc


def _sc_gather(table, idx):
    B = idx.shape[0]
    V, D = table.shape
    info = plsc.get_sparse_core_info()
    nw = info.num_cores * info.num_subcores
    b_per_w = B // nw
    mesh = plsc.VectorSubcoreMesh(core_axis_name="c", subcore_axis_name="s")

    @functools.partial(
        pl.kernel,
        mesh=mesh,
        out_type=jax.ShapeDtypeStruct((B, D), jnp.float32),
        scratch_types=[
            pltpu.VMEM((b_per_w,), jnp.int32),
            pltpu.VMEM((b_per_w, D), jnp.float32),
            pltpu.SemaphoreType.DMA,
        ],
    )
    def gather_kernel(table_hbm, idx_hbm, out_hbm, idx_v, rows_v, sem):
        wid = lax.axis_index("s") * info.num_cores + lax.axis_index("c")
        base = wid * b_per_w
        pltpu.sync_copy(idx_hbm.at[pl.ds(base, b_per_w)], idx_v)
        pltpu.async_copy(table_hbm.at[idx_v], rows_v, sem).wait()
        pltpu.sync_copy(rows_v, out_hbm.at[pl.ds(base, b_per_w)])

    return gather_kernel(table, idx)


def _tc_project_t(embeds, W_out, b_out, tile_v=2048):
    B, D = embeds.shape
    V = W_out.shape[0]
    nv = pl.cdiv(V, tile_v)

    def body(e_ref, w_ref, b_ref, o_ref):
        acc = lax.dot_general(
            w_ref[...], e_ref[...],
            dimension_numbers=(((1,), (1,)), ((), ())),
            preferred_element_type=jnp.float32,
        )
        o_ref[...] = acc + b_ref[...][:, None]

    return pl.pallas_call(
        body,
        grid=(nv,),
        in_specs=[
            pl.BlockSpec((B, D), lambda i: (0, 0)),
            pl.BlockSpec((tile_v, D), lambda i: (i, 0)),
            pl.BlockSpec((tile_v,), lambda i: (i,)),
        ],
        out_specs=pl.BlockSpec((tile_v, B), lambda i: (i, 0)),
        out_shape=jax.ShapeDtypeStruct((V, B), jnp.float32),
    )(embeds, W_out, b_out)


def kernel(input_word_ids, in_table, W_out, b_out):
    ids = input_word_ids.astype(jnp.int32)
    embeds = _sc_gather(in_table, ids)
    return _tc_project_t(embeds, W_out, b_out).T


# speedup vs baseline: 3.2023x; 1.0183x over previous
import functools

import jax
import jax.numpy as jnp
from jax import lax
from jax.experimental import pallas as pl
from jax.experimental.pallas import tpu as pltpu
from jax.experimental.pallas import tpu_sc as plsc


def _sc_gather(table, idx):
    B = idx.shape[0]
    V, D = table.shape
    info = plsc.get_sparse_core_info()
    nw = info.num_cores * info.num_subcores
    b_per_w = B // nw
    mesh = plsc.VectorSubcoreMesh(core_axis_name="c", subcore_axis_name="s")

    @functools.partial(
        pl.kernel,
        mesh=mesh,
        out_type=jax.ShapeDtypeStruct((B, D), jnp.float32),
        scratch_types=[
            pltpu.VMEM((b_per_w,), jnp.int32),
            pltpu.VMEM((b_per_w, D), jnp.float32),
            pltpu.SemaphoreType.DMA,
        ],
    )
    def gather_kernel(table_hbm, idx_hbm, out_hbm, idx_v, rows_v, sem):
        wid = lax.axis_index("s") * info.num_cores + lax.axis_index("c")
        base = wid * b_per_w
        pltpu.sync_copy(idx_hbm.at[pl.ds(base, b_per_w)], idx_v)
        pltpu.async_copy(table_hbm.at[idx_v], rows_v, sem).wait()
        pltpu.sync_copy(rows_v, out_hbm.at[pl.ds(base, b_per_w)])

    return gather_kernel(table, idx)


def _tc_project_t(embeds, W_out, b_out, tile_v=4096):
    B, D = embeds.shape
    V = W_out.shape[0]
    nv = pl.cdiv(V, tile_v)

    def body(e_ref, w_ref, b_ref, o_ref):
        acc = lax.dot_general(
            w_ref[...], e_ref[...],
            dimension_numbers=(((1,), (1,)), ((), ())),
            preferred_element_type=jnp.float32,
        )
        o_ref[...] = acc + b_ref[...][:, None]

    return pl.pallas_call(
        body,
        grid=(nv,),
        in_specs=[
            pl.BlockSpec((B, D), lambda i: (0, 0)),
            pl.BlockSpec((tile_v, D), lambda i: (i, 0)),
            pl.BlockSpec((tile_v,), lambda i: (i,)),
        ],
        out_specs=pl.BlockSpec((tile_v, B), lambda i: (i, 0)),
        out_shape=jax.ShapeDtypeStruct((V, B), jnp.float32),
    )(embeds, W_out, b_out)


def kernel(input_word_ids, in_table, W_out, b_out):
    ids = input_word_ids.astype(jnp.int32)
    embeds = _sc_gather(in_table, ids)
    return _tc_project_t(embeds, W_out, b_out).T
